# baseline (speedup 1.0000x reference)
import functools

import jax
import jax.numpy as jnp
from jax import lax
from jax.experimental import pallas as pl
from jax.experimental.pallas import tpu as pltpu
from jax.experimental.pallas import tpu_sc as plsc

N_NODES = 10000
D = 256
N_EDGES = 160000

NC = 2
NS = 16
NW = NC * NS
OWN = 320
TRASH = OWN
ACC_ROWS = 328
B = 128
NBND = 48


def _segsum_body(bnd_hbm, src_hbm, dst_hbm, x_hbm, out_hbm,
                 bndb, srcbuf, dstbuf, rows, acc):
    cid = lax.axis_index("c")
    sid = lax.axis_index("s")
    w = sid * NC + cid
    lo_row = w * OWN

    zv = jnp.zeros((16,), jnp.float32)

    def zrow(r, _):
        for f in range(D // 16):
            acc[r, pl.ds(f * 16, 16)] = zv
        return 0

    lax.fori_loop(0, ACC_ROWS, zrow, 0)

    pltpu.sync_copy(bnd_hbm, bndb)
    bv = bndb[pl.ds(w, 16)]
    lo = bv[0]
    hi = bv[1]
    k0 = lo // B
    k1 = (hi + B - 1) // B

    def block(k, _):
        base = pl.multiple_of(k * B, B)
        pltpu.sync_copy(src_hbm.at[pl.ds(base, B)], srcbuf)
        pltpu.sync_copy(dst_hbm.at[pl.ds(base, B)], dstbuf.at[pl.ds(0, B)])
        pltpu.sync_copy(x_hbm.at[srcbuf], rows)

        def edge(e, _):
            d = dstbuf[pl.ds(e, 16)][0]
            local = d - lo_row
            r = jnp.where((local >= 0) & (local < OWN), local, TRASH)
            for f in range(D // 16):
                sl = pl.ds(f * 16, 16)
                plsc.addupdate(acc.at[r, sl], rows[e, sl])
            return 0

        return lax.fori_loop(0, B, edge, 0)

    lax.fori_loop(k0, k1, block, 0, unroll=False)

    gbase = pl.multiple_of(w * OWN, 64)

    @pl.when(w < NW - 1)
    def _():
        pltpu.sync_copy(acc.at[pl.ds(0, OWN)], out_hbm.at[pl.ds(gbase, OWN)])

    @pl.when(w == NW - 1)
    def _():
        pltpu.sync_copy(acc.at[pl.ds(0, N_NODES - (NW - 1) * OWN)],
                        out_hbm.at[pl.ds(gbase, N_NODES - (NW - 1) * OWN)])


@jax.jit
def _sc_segsum(bnd, ssrc, sdst, x):
    mesh = plsc.VectorSubcoreMesh(core_axis_name="c", subcore_axis_name="s")
    f = pl.kernel(
        _segsum_body,
        out_type=jax.ShapeDtypeStruct((N_NODES, D), jnp.float32),
        mesh=mesh,
        scratch_types=[
            pltpu.VMEM((NBND,), jnp.int32),
            pltpu.VMEM((B,), jnp.int32),
            pltpu.VMEM((B + 16,), jnp.int32),
            pltpu.VMEM((B, D), jnp.float32),
            pltpu.VMEM((ACC_ROWS, D), jnp.float32),
        ],
    )
    return f(bnd, ssrc, sdst, x)


def _dense_body(apply_relu, agg_ref, x_ref, wl_ref, wr_ref, b_ref, o_ref):
    h = jnp.dot(agg_ref[...], wl_ref[...], preferred_element_type=jnp.float32)
    h = h + jnp.dot(x_ref[...], wr_ref[...], preferred_element_type=jnp.float32)
    h = h + b_ref[...]
    nrm = jnp.sqrt(jnp.sum(h * h, axis=1, keepdims=True))
    h = h / jnp.maximum(nrm, 1e-12)
    if apply_relu:
        h = jnp.maximum(h, 0.0)
    o_ref[...] = h


def _dense(agg, x, wlT, wrT, b2d, apply_relu):
    R = 1000
    grid = (N_NODES // R,)
    return pl.pallas_call(
        functools.partial(_dense_body, apply_relu),
        grid=grid,
        in_specs=[
            pl.BlockSpec((R, D), lambda i: (i, 0)),
            pl.BlockSpec((R, D), lambda i: (i, 0)),
            pl.BlockSpec((D, D), lambda i: (0, 0)),
            pl.BlockSpec((D, D), lambda i: (0, 0)),
            pl.BlockSpec((1, D), lambda i: (0, 0)),
        ],
        out_specs=pl.BlockSpec((R, D), lambda i: (i, 0)),
        out_shape=jax.ShapeDtypeStruct((N_NODES, D), jnp.float32),
    )(agg, x, wlT, wrT, b2d)


def kernel(x, edge_index, W1_l, b1, W1_r, W2_l, b2, W2_r, W3_l, b3, W3_r):
    src = edge_index[0].astype(jnp.int32)
    dst = edge_index[1].astype(jnp.int32)
    sdst, ssrc = lax.sort_key_val(dst, src)
    bounds = jnp.searchsorted(sdst, jnp.arange(NW + 1, dtype=jnp.int32) * OWN)
    bnd = jnp.zeros((NBND,), jnp.int32).at[: NW + 1].set(
        bounds.astype(jnp.int32))

    agg1 = _sc_segsum(bnd, ssrc, sdst, x)
    h1 = _dense(agg1, x, W1_l.T, W1_r.T, b1.reshape(1, D), True)
    agg2 = _sc_segsum(bnd, ssrc, sdst, h1)
    h2 = _dense(agg2, h1, W2_l.T, W2_r.T, b2.reshape(1, D), True)
    agg3 = _sc_segsum(bnd, ssrc, sdst, h2)
    out = _dense(agg3, h2, W3_l.T, W3_r.T, b3.reshape(1, D), False)
    return out
